# baseline (speedup 1.0000x reference)
import functools

import jax
import jax.numpy as jnp
from jax import lax
from jax.experimental import pallas as pl
from jax.experimental.pallas import tpu as pltpu
from jax.experimental.pallas import tpu_sc as plsc

N = 10000
D = 256
DH = 128
NT = 16
EB = 64
E_PAD = 163840
EPT = E_PAD // NT
NB = EPT // EB
RPT = 632
RPT_LAST = N - 15 * RPT

_sc_mesh = plsc.VectorSubcoreMesh(core_axis_name="c", subcore_axis_name="s")


def _deg_body(zer_hbm, dst_hbm, ew_hbm, deg_hbm, dacc, idxv, ewv):
    cid = lax.axis_index("c")
    sid = lax.axis_index("s")

    @pl.when(cid == 0)
    def _():
        @pl.when(sid == 0)
        def _():
            pltpu.sync_copy(zer_hbm, dacc)

        plsc.subcore_barrier()

        def body(g, carry):
            off = sid * EPT + g * EB
            pltpu.sync_copy(dst_hbm.at[pl.ds(off, EB)], idxv)
            pltpu.sync_copy(ew_hbm.at[pl.ds(off, EB)], ewv)
            pltpu.sync_copy(ewv, dacc.at[idxv], add=True)
            return carry

        lax.fori_loop(0, NB, body, 0)
        plsc.subcore_barrier()

        @pl.when(sid == 0)
        def _():
            pltpu.sync_copy(dacc, deg_hbm)


_deg_kernel = functools.partial(
    pl.kernel,
    out_type=jax.ShapeDtypeStruct((N,), jnp.float32),
    mesh=_sc_mesh,
    scratch_types=[
        pltpu.VMEM_SHARED((N,), jnp.float32),
        pltpu.VMEM((EB,), jnp.int32),
        pltpu.VMEM((EB,), jnp.float32),
    ],
)(_deg_body)


def _mp_body(hA_hbm, hB_hbm, src_hbm, dst_hbm, ewrep_hbm, sA_hbm,
             sB_hbm, acc, idx0, idx1, dst0, dst1, ew0, ew1, rows0, rows1,
             sem0, sem1):
    cid = lax.axis_index("c")
    sid = lax.axis_index("s")
    bufs = ((idx0, dst0, ew0, rows0, sem0), (idx1, dst1, ew1, rows1, sem1))

    base = pl.multiple_of(sid * RPT, 8)

    def zrow(r, c2):
        for k in range(8):
            rows0[r, pl.ds(k * 16, 16)] = jnp.zeros((16,), jnp.float32)
        return c2

    lax.fori_loop(0, EB, zrow, 0)
    for j in range(8):
        pltpu.sync_copy(rows0, acc.at[pl.ds(base + j * EB, EB)])

    @pl.when(sid < 15)
    def _():
        pltpu.sync_copy(rows0, acc.at[pl.ds(base + 8 * EB, EB)])
        pltpu.sync_copy(rows0.at[pl.ds(0, RPT - 9 * EB)],
                        acc.at[pl.ds(base + 9 * EB, RPT - 9 * EB)])

    @pl.when(sid == 15)
    def _():
        pltpu.sync_copy(rows0.at[pl.ds(0, RPT_LAST - 8 * EB)],
                        acc.at[pl.ds(15 * RPT + 8 * EB, RPT_LAST - 8 * EB)])

    plsc.subcore_barrier()

    def stage_issue(g, b):
        idxb, dstb, ewb, rowsb, semb = bufs[b]
        off = sid * EPT + g * EB
        pltpu.sync_copy(src_hbm.at[pl.ds(off, EB)], idxb)
        pltpu.sync_copy(dst_hbm.at[pl.ds(off, EB)], dstb)
        pltpu.sync_copy(ewrep_hbm.at[pl.ds(off, EB)], ewb)

        @pl.when(cid == 0)
        def _():
            pltpu.async_copy(hA_hbm.at[idxb], rowsb, semb)

        @pl.when(cid == 1)
        def _():
            pltpu.async_copy(hB_hbm.at[idxb], rowsb, semb)

    def segment(g, b):
        idxb, dstb, ewb, rowsb, semb = bufs[b]

        @pl.when(g + 1 < NB)
        def _():
            stage_issue(g + 1, 1 - b)

        pltpu.make_async_copy(hA_hbm.at[pl.ds(0, EB)], rowsb, semb).wait()

        def scale(r, c2):
            ewr = ewb[r, :]
            for k in range(8):
                sl = pl.ds(k * 16, 16)
                rowsb[r, sl] = rowsb[r, sl] * ewr
            return c2

        lax.fori_loop(0, EB, scale, 0)
        pltpu.sync_copy(rowsb, acc.at[dstb], add=True)

    stage_issue(0, 0)

    def body(p, carry):
        segment(2 * p, 0)
        segment(2 * p + 1, 1)
        return carry

    lax.fori_loop(0, NB // 2, body, 0)
    plsc.subcore_barrier()

    @pl.when((cid == 0) & (sid < 15))
    def _():
        pltpu.sync_copy(acc.at[pl.ds(base, RPT)], sA_hbm.at[pl.ds(base, RPT)])

    @pl.when((cid == 0) & (sid == 15))
    def _():
        pltpu.sync_copy(acc.at[pl.ds(15 * RPT, RPT_LAST)],
                        sA_hbm.at[pl.ds(15 * RPT, RPT_LAST)])

    @pl.when((cid == 1) & (sid < 15))
    def _():
        pltpu.sync_copy(acc.at[pl.ds(base, RPT)], sB_hbm.at[pl.ds(base, RPT)])

    @pl.when((cid == 1) & (sid == 15))
    def _():
        pltpu.sync_copy(acc.at[pl.ds(15 * RPT, RPT_LAST)],
                        sB_hbm.at[pl.ds(15 * RPT, RPT_LAST)])


_mp_kernel = functools.partial(
    pl.kernel,
    out_type=[jax.ShapeDtypeStruct((N, DH), jnp.float32),
              jax.ShapeDtypeStruct((N, DH), jnp.float32)],
    mesh=_sc_mesh,
    scratch_types=[
        pltpu.VMEM_SHARED((N, DH), jnp.float32),
        pltpu.VMEM((EB,), jnp.int32),
        pltpu.VMEM((EB,), jnp.int32),
        pltpu.VMEM((EB,), jnp.int32),
        pltpu.VMEM((EB,), jnp.int32),
        pltpu.VMEM((EB, 16), jnp.float32),
        pltpu.VMEM((EB, 16), jnp.float32),
        pltpu.VMEM((EB, DH), jnp.float32),
        pltpu.VMEM((EB, DH), jnp.float32),
        pltpu.SemaphoreType.DMA,
        pltpu.SemaphoreType.DMA,
    ],
)(_mp_body)


RB = 400


def _dinv(deg_blk):
    d = deg_blk + 1.0
    return jnp.where(d > 0, lax.rsqrt(d), 0.0)


def _mm1_body(x_ref, w_ref, deg_ref, hA_ref, hB_ref):
    dinv = _dinv(deg_ref[...])
    h = jnp.dot(x_ref[...], w_ref[...],
                preferred_element_type=jnp.float32) * dinv
    hA_ref[...] = h[:, :DH]
    hB_ref[...] = h[:, DH:]


def _mm2_body(sA_ref, sB_ref, hA_ref, hB_ref, deg_ref, b1_ref, w2_ref,
              h2A_ref, h2B_ref):
    dinv = _dinv(deg_ref[...])
    o = jnp.concatenate(
        [sA_ref[...] + hA_ref[...], sB_ref[...] + hB_ref[...]], axis=1)
    o = jnp.maximum(o * dinv + b1_ref[...], 0.0)
    h2 = jnp.dot(o, w2_ref[...], preferred_element_type=jnp.float32) * dinv
    h2A_ref[...] = h2[:, :DH]
    h2B_ref[...] = h2[:, DH:]


def _mm3_body(sA_ref, sB_ref, hA_ref, hB_ref, deg_ref, b2_ref, out_ref):
    dinv = _dinv(deg_ref[...])
    o = jnp.concatenate(
        [sA_ref[...] + hA_ref[...], sB_ref[...] + hB_ref[...]], axis=1)
    out_ref[...] = o * dinv + b2_ref[...]


_half = pl.BlockSpec((RB, DH), lambda i: (i, 0))
_full = pl.BlockSpec((RB, D), lambda i: (i, 0))
_degs = pl.BlockSpec((RB, 1), lambda i: (i, 0))
_wspec = pl.BlockSpec((D, D), lambda i: (0, 0))
_bspec = pl.BlockSpec((1, D), lambda i: (0, 0))

_mm1 = pl.pallas_call(
    _mm1_body,
    grid=(N // RB,),
    in_specs=[_full, _wspec, _degs],
    out_specs=[_half, _half],
    out_shape=[jax.ShapeDtypeStruct((N, DH), jnp.float32),
               jax.ShapeDtypeStruct((N, DH), jnp.float32)],
)

_mm2 = pl.pallas_call(
    _mm2_body,
    grid=(N // RB,),
    in_specs=[_half, _half, _half, _half, _degs, _bspec, _wspec],
    out_specs=[_half, _half],
    out_shape=[jax.ShapeDtypeStruct((N, DH), jnp.float32),
               jax.ShapeDtypeStruct((N, DH), jnp.float32)],
)

_mm3 = pl.pallas_call(
    _mm3_body,
    grid=(N // RB,),
    in_specs=[_half, _half, _half, _half, _degs, _bspec],
    out_specs=_full,
    out_shape=jax.ShapeDtypeStruct((N, D), jnp.float32),
)


@jax.jit
def kernel(x, edge_index, edge_weight, W1, b1, W2, b2):
    src = edge_index[0].astype(jnp.int32)
    dst = edge_index[1].astype(jnp.int32)
    src, dst, ew_s = lax.sort((src, dst, edge_weight), num_keys=1)
    pad = E_PAD - src.shape[0]
    src = jnp.concatenate([src, jnp.zeros((pad,), jnp.int32)])
    dst = jnp.concatenate([dst, jnp.zeros((pad,), jnp.int32)])
    ew = jnp.concatenate([ew_s, jnp.zeros((pad,), jnp.float32)])

    zer1 = jnp.zeros((N,), jnp.float32)

    ew_rep = jnp.broadcast_to(ew[:, None], (E_PAD, 16))

    deg = _deg_kernel(zer1, dst, ew).reshape(N, 1)
    b1r = b1.reshape(1, D)
    b2r = b2.reshape(1, D)

    hA, hB = _mm1(x, W1, deg)
    sA, sB = _mp_kernel(hA, hB, src, dst, ew_rep)
    h2A, h2B = _mm2(sA, sB, hA, hB, deg, b1r, W2)
    s2A, s2B = _mp_kernel(h2A, h2B, src, dst, ew_rep)
    return _mm3(s2A, s2B, h2A, h2B, deg, b2r)


# speedup vs baseline: 1.2289x; 1.2289x over previous
import functools

import jax
import jax.numpy as jnp
from jax import lax
from jax.experimental import pallas as pl
from jax.experimental.pallas import tpu as pltpu
from jax.experimental.pallas import tpu_sc as plsc

N = 10000
D = 256
DH = 128
NT = 16
EB = 64
E_PAD = 163840
EPT = E_PAD // NT
NB = EPT // EB
RPT = 632
RPT_LAST = N - 15 * RPT

_sc_mesh = plsc.VectorSubcoreMesh(core_axis_name="c", subcore_axis_name="s")


def _deg_body(zer_hbm, dst_hbm, ew_hbm, degA_hbm, degB_hbm, dacc, idxv, ewv):
    cid = lax.axis_index("c")
    sid = lax.axis_index("s")

    @pl.when(sid == 0)
    def _():
        pltpu.sync_copy(zer_hbm, dacc)

    plsc.subcore_barrier()

    def body(g, carry):
        off = cid * (E_PAD // 2) + sid * (EPT // 2) + g * EB
        pltpu.sync_copy(dst_hbm.at[pl.ds(off, EB)], idxv)
        pltpu.sync_copy(ew_hbm.at[pl.ds(off, EB)], ewv)
        pltpu.sync_copy(ewv, dacc.at[idxv], add=True)
        return carry

    lax.fori_loop(0, NB // 2, body, 0)
    plsc.subcore_barrier()

    @pl.when((cid == 0) & (sid == 0))
    def _():
        pltpu.sync_copy(dacc, degA_hbm)

    @pl.when((cid == 1) & (sid == 0))
    def _():
        pltpu.sync_copy(dacc, degB_hbm)


_deg_kernel = functools.partial(
    pl.kernel,
    out_type=[jax.ShapeDtypeStruct((N,), jnp.float32),
              jax.ShapeDtypeStruct((N,), jnp.float32)],
    mesh=_sc_mesh,
    scratch_types=[
        pltpu.VMEM_SHARED((N,), jnp.float32),
        pltpu.VMEM((EB,), jnp.int32),
        pltpu.VMEM((EB,), jnp.float32),
    ],
)(_deg_body)


def _mp_body(hA_hbm, hB_hbm, src_hbm, dst_hbm, ewrep_hbm, sA_hbm,
             sB_hbm, acc, idx0, idx1, dst0, dst1, ew0, ew1, rows0, rows1,
             sem0, sem1):
    cid = lax.axis_index("c")
    sid = lax.axis_index("s")
    bufs = ((idx0, dst0, ew0, rows0, sem0), (idx1, dst1, ew1, rows1, sem1))

    base = pl.multiple_of(sid * RPT, 8)

    def zrow(r, c2):
        for k in range(8):
            rows0[r, pl.ds(k * 16, 16)] = jnp.zeros((16,), jnp.float32)
        return c2

    lax.fori_loop(0, EB, zrow, 0)
    for j in range(8):
        pltpu.sync_copy(rows0, acc.at[pl.ds(base + j * EB, EB)])

    @pl.when(sid < 15)
    def _():
        pltpu.sync_copy(rows0, acc.at[pl.ds(base + 8 * EB, EB)])
        pltpu.sync_copy(rows0.at[pl.ds(0, RPT - 9 * EB)],
                        acc.at[pl.ds(base + 9 * EB, RPT - 9 * EB)])

    @pl.when(sid == 15)
    def _():
        pltpu.sync_copy(rows0.at[pl.ds(0, RPT_LAST - 8 * EB)],
                        acc.at[pl.ds(15 * RPT + 8 * EB, RPT_LAST - 8 * EB)])

    plsc.subcore_barrier()

    def stage_issue(g, b):
        idxb, dstb, ewb, rowsb, semb = bufs[b]
        off = sid * EPT + g * EB
        pltpu.sync_copy(src_hbm.at[pl.ds(off, EB)], idxb)
        pltpu.sync_copy(dst_hbm.at[pl.ds(off, EB)], dstb)
        pltpu.sync_copy(ewrep_hbm.at[pl.ds(off, EB)], ewb)

        @pl.when(cid == 0)
        def _():
            pltpu.async_copy(hA_hbm.at[idxb], rowsb, semb)

        @pl.when(cid == 1)
        def _():
            pltpu.async_copy(hB_hbm.at[idxb], rowsb, semb)

    def segment(g, b):
        idxb, dstb, ewb, rowsb, semb = bufs[b]

        @pl.when(g + 1 < NB)
        def _():
            stage_issue(g + 1, 1 - b)

        pltpu.make_async_copy(hA_hbm.at[pl.ds(0, EB)], rowsb, semb).wait()

        def scale(r, c2):
            ewr = ewb[r, :]
            for k in range(8):
                sl = pl.ds(k * 16, 16)
                rowsb[r, sl] = rowsb[r, sl] * ewr
            return c2

        lax.fori_loop(0, EB, scale, 0)
        pltpu.sync_copy(rowsb, acc.at[dstb], add=True)

    stage_issue(0, 0)

    def body(p, carry):
        segment(2 * p, 0)
        segment(2 * p + 1, 1)
        return carry

    lax.fori_loop(0, NB // 2, body, 0)
    plsc.subcore_barrier()

    @pl.when((cid == 0) & (sid < 15))
    def _():
        pltpu.sync_copy(acc.at[pl.ds(base, RPT)], sA_hbm.at[pl.ds(base, RPT)])

    @pl.when((cid == 0) & (sid == 15))
    def _():
        pltpu.sync_copy(acc.at[pl.ds(15 * RPT, RPT_LAST)],
                        sA_hbm.at[pl.ds(15 * RPT, RPT_LAST)])

    @pl.when((cid == 1) & (sid < 15))
    def _():
        pltpu.sync_copy(acc.at[pl.ds(base, RPT)], sB_hbm.at[pl.ds(base, RPT)])

    @pl.when((cid == 1) & (sid == 15))
    def _():
        pltpu.sync_copy(acc.at[pl.ds(15 * RPT, RPT_LAST)],
                        sB_hbm.at[pl.ds(15 * RPT, RPT_LAST)])


_mp_kernel = functools.partial(
    pl.kernel,
    out_type=[jax.ShapeDtypeStruct((N, DH), jnp.float32),
              jax.ShapeDtypeStruct((N, DH), jnp.float32)],
    mesh=_sc_mesh,
    scratch_types=[
        pltpu.VMEM_SHARED((N, DH), jnp.float32),
        pltpu.VMEM((EB,), jnp.int32),
        pltpu.VMEM((EB,), jnp.int32),
        pltpu.VMEM((EB,), jnp.int32),
        pltpu.VMEM((EB,), jnp.int32),
        pltpu.VMEM((EB, 16), jnp.float32),
        pltpu.VMEM((EB, 16), jnp.float32),
        pltpu.VMEM((EB, DH), jnp.float32),
        pltpu.VMEM((EB, DH), jnp.float32),
        pltpu.SemaphoreType.DMA,
        pltpu.SemaphoreType.DMA,
    ],
)(_mp_body)


RB = 400


def _dinv(deg_blk):
    d = deg_blk + 1.0
    return jnp.where(d > 0, lax.rsqrt(d), 0.0)


def _mm1_body(x_ref, w_ref, yA_ref, yB_ref):
    h = jnp.dot(x_ref[...], w_ref[...], preferred_element_type=jnp.float32)
    yA_ref[...] = h[:, :DH]
    yB_ref[...] = h[:, DH:]


def _scale1_body(yA_ref, yB_ref, dA_ref, dB_ref, hA_ref, hB_ref, dinv_ref):
    dinv = _dinv(dA_ref[...] + dB_ref[...])
    hA_ref[...] = yA_ref[...] * dinv
    hB_ref[...] = yB_ref[...] * dinv
    dinv_ref[...] = dinv


def _mm2_body(sA_ref, sB_ref, hA_ref, hB_ref, dinv_ref, b1_ref, w2_ref,
              h2A_ref, h2B_ref):
    dinv = dinv_ref[...]
    o = jnp.concatenate(
        [sA_ref[...] + hA_ref[...], sB_ref[...] + hB_ref[...]], axis=1)
    o = jnp.maximum(o * dinv + b1_ref[...], 0.0)
    h2 = jnp.dot(o, w2_ref[...], preferred_element_type=jnp.float32) * dinv
    h2A_ref[...] = h2[:, :DH]
    h2B_ref[...] = h2[:, DH:]


def _mm3_body(sA_ref, sB_ref, hA_ref, hB_ref, dinv_ref, b2_ref, out_ref):
    o = jnp.concatenate(
        [sA_ref[...] + hA_ref[...], sB_ref[...] + hB_ref[...]], axis=1)
    out_ref[...] = o * dinv_ref[...] + b2_ref[...]


_half = pl.BlockSpec((RB, DH), lambda i: (i, 0))
_full = pl.BlockSpec((RB, D), lambda i: (i, 0))
_degs = pl.BlockSpec((RB, 1), lambda i: (i, 0))
_wspec = pl.BlockSpec((D, D), lambda i: (0, 0))
_bspec = pl.BlockSpec((1, D), lambda i: (0, 0))

_mm1 = pl.pallas_call(
    _mm1_body,
    grid=(N // RB,),
    in_specs=[_full, _wspec],
    out_specs=[_half, _half],
    out_shape=[jax.ShapeDtypeStruct((N, DH), jnp.float32),
               jax.ShapeDtypeStruct((N, DH), jnp.float32)],
)

_scale1 = pl.pallas_call(
    _scale1_body,
    grid=(N // RB,),
    in_specs=[_half, _half, _degs, _degs],
    out_specs=[_half, _half, _degs],
    out_shape=[jax.ShapeDtypeStruct((N, DH), jnp.float32),
               jax.ShapeDtypeStruct((N, DH), jnp.float32),
               jax.ShapeDtypeStruct((N, 1), jnp.float32)],
)

_mm2 = pl.pallas_call(
    _mm2_body,
    grid=(N // RB,),
    in_specs=[_half, _half, _half, _half, _degs, _bspec, _wspec],
    out_specs=[_half, _half],
    out_shape=[jax.ShapeDtypeStruct((N, DH), jnp.float32),
               jax.ShapeDtypeStruct((N, DH), jnp.float32)],
)

_mm3 = pl.pallas_call(
    _mm3_body,
    grid=(N // RB,),
    in_specs=[_half, _half, _half, _half, _degs, _bspec],
    out_specs=_full,
    out_shape=jax.ShapeDtypeStruct((N, D), jnp.float32),
)


@jax.jit
def kernel(x, edge_index, edge_weight, W1, b1, W2, b2):
    src = edge_index[0].astype(jnp.int32)
    dst = edge_index[1].astype(jnp.int32)
    pad = E_PAD - src.shape[0]
    src = jnp.concatenate([src, jnp.zeros((pad,), jnp.int32)])
    dst = jnp.concatenate([dst, jnp.zeros((pad,), jnp.int32)])
    ew = jnp.concatenate([edge_weight, jnp.zeros((pad,), jnp.float32)])

    zer1 = jnp.zeros((N,), jnp.float32)

    ew_rep = jnp.broadcast_to(ew[:, None], (E_PAD, 16))

    degA, degB = _deg_kernel(zer1, dst, ew)
    b1r = b1.reshape(1, D)
    b2r = b2.reshape(1, D)

    yA, yB = _mm1(x, W1)
    hA, hB, dinv = _scale1(yA, yB, degA.reshape(N, 1), degB.reshape(N, 1))
    sA, sB = _mp_kernel(hA, hB, src, dst, ew_rep)
    h2A, h2B = _mm2(sA, sB, hA, hB, dinv, b1r, W2)
    s2A, s2B = _mp_kernel(h2A, h2B, src, dst, ew_rep)
    return _mm3(s2A, s2B, h2A, h2B, dinv, b2r)


# speedup vs baseline: 1.3371x; 1.0880x over previous
import functools

import jax
import jax.numpy as jnp
from jax import lax
from jax.experimental import pallas as pl
from jax.experimental.pallas import tpu as pltpu
from jax.experimental.pallas import tpu_sc as plsc

N = 10000
D = 256
DH = 128
NT = 16
EB = 64
E_PAD = 163840
EPT = E_PAD // NT
NB = EPT // EB
RPT = 632
RPT_LAST = N - 15 * RPT

_sc_mesh = plsc.VectorSubcoreMesh(core_axis_name="c", subcore_axis_name="s")


def _deg_body(zer_hbm, dst_hbm, ew_hbm, degA_hbm, degB_hbm, dacc, idxv, ewv):
    cid = lax.axis_index("c")
    sid = lax.axis_index("s")

    @pl.when(sid == 0)
    def _():
        pltpu.sync_copy(zer_hbm, dacc)

    plsc.subcore_barrier()

    def body(g, carry):
        off = cid * (E_PAD // 2) + sid * (EPT // 2) + g * EB
        pltpu.sync_copy(dst_hbm.at[pl.ds(off, EB)], idxv)
        pltpu.sync_copy(ew_hbm.at[pl.ds(off, EB)], ewv)
        pltpu.sync_copy(ewv, dacc.at[idxv], add=True)
        return carry

    lax.fori_loop(0, NB // 2, body, 0)
    plsc.subcore_barrier()

    @pl.when((cid == 0) & (sid == 0))
    def _():
        pltpu.sync_copy(dacc, degA_hbm)

    @pl.when((cid == 1) & (sid == 0))
    def _():
        pltpu.sync_copy(dacc, degB_hbm)


_deg_kernel = functools.partial(
    pl.kernel,
    out_type=[jax.ShapeDtypeStruct((N,), jnp.float32),
              jax.ShapeDtypeStruct((N,), jnp.float32)],
    mesh=_sc_mesh,
    scratch_types=[
        pltpu.VMEM_SHARED((N,), jnp.float32),
        pltpu.VMEM((EB,), jnp.int32),
        pltpu.VMEM((EB,), jnp.float32),
    ],
)(_deg_body)


def _mp_body(hA_hbm, hB_hbm, src_hbm, dst_hbm, ewrep_hbm, sA_hbm,
             sB_hbm, acc, idx0, idx1, dst0, dst1, ew0, ew1, rows0, rows1,
             sem0, sem1):
    cid = lax.axis_index("c")
    sid = lax.axis_index("s")
    bufs = ((idx0, dst0, ew0, rows0, sem0), (idx1, dst1, ew1, rows1, sem1))

    base = pl.multiple_of(sid * RPT, 8)

    def zrow(r, c2):
        for k in range(8):
            rows0[r, pl.ds(k * 16, 16)] = jnp.zeros((16,), jnp.float32)
        return c2

    lax.fori_loop(0, EB, zrow, 0)
    for j in range(8):
        pltpu.sync_copy(rows0, acc.at[pl.ds(base + j * EB, EB)])

    @pl.when(sid < 15)
    def _():
        pltpu.sync_copy(rows0, acc.at[pl.ds(base + 8 * EB, EB)])
        pltpu.sync_copy(rows0.at[pl.ds(0, RPT - 9 * EB)],
                        acc.at[pl.ds(base + 9 * EB, RPT - 9 * EB)])

    @pl.when(sid == 15)
    def _():
        pltpu.sync_copy(rows0.at[pl.ds(0, RPT_LAST - 8 * EB)],
                        acc.at[pl.ds(15 * RPT + 8 * EB, RPT_LAST - 8 * EB)])

    plsc.subcore_barrier()

    def stage_issue(g, b):
        idxb, dstb, ewb, rowsb, semb = bufs[b]
        off = sid * EPT + g * EB
        pltpu.sync_copy(src_hbm.at[pl.ds(off, EB)], idxb)
        pltpu.sync_copy(dst_hbm.at[pl.ds(off, EB)], dstb)
        pltpu.sync_copy(ewrep_hbm.at[pl.ds(off, EB)], ewb)

        @pl.when(cid == 0)
        def _():
            pltpu.async_copy(hA_hbm.at[idxb], rowsb, semb)

        @pl.when(cid == 1)
        def _():
            pltpu.async_copy(hB_hbm.at[idxb], rowsb, semb)

    def segment(g, b):
        idxb, dstb, ewb, rowsb, semb = bufs[b]

        @pl.when(g + 1 < NB)
        def _():
            stage_issue(g + 1, 1 - b)

        pltpu.make_async_copy(hA_hbm.at[pl.ds(0, EB)], rowsb, semb).wait()

        def scale(r, c2):
            ewr = ewb[r, :]
            for k in range(8):
                sl = pl.ds(k * 16, 16)
                rowsb[r, sl] = rowsb[r, sl] * ewr
            return c2

        lax.fori_loop(0, EB, scale, 0)
        pltpu.sync_copy(rowsb, acc.at[dstb], add=True)

    stage_issue(0, 0)

    def body(p, carry):
        segment(2 * p, 0)
        segment(2 * p + 1, 1)
        return carry

    lax.fori_loop(0, NB // 2, body, 0)
    plsc.subcore_barrier()

    @pl.when((cid == 0) & (sid < 15))
    def _():
        pltpu.sync_copy(acc.at[pl.ds(base, RPT)], sA_hbm.at[pl.ds(base, RPT)])

    @pl.when((cid == 0) & (sid == 15))
    def _():
        pltpu.sync_copy(acc.at[pl.ds(15 * RPT, RPT_LAST)],
                        sA_hbm.at[pl.ds(15 * RPT, RPT_LAST)])

    @pl.when((cid == 1) & (sid < 15))
    def _():
        pltpu.sync_copy(acc.at[pl.ds(base, RPT)], sB_hbm.at[pl.ds(base, RPT)])

    @pl.when((cid == 1) & (sid == 15))
    def _():
        pltpu.sync_copy(acc.at[pl.ds(15 * RPT, RPT_LAST)],
                        sB_hbm.at[pl.ds(15 * RPT, RPT_LAST)])


_mp_kernel = functools.partial(
    pl.kernel,
    out_type=[jax.ShapeDtypeStruct((N, DH), jnp.float32),
              jax.ShapeDtypeStruct((N, DH), jnp.float32)],
    mesh=_sc_mesh,
    scratch_types=[
        pltpu.VMEM_SHARED((N, DH), jnp.float32),
        pltpu.VMEM((EB,), jnp.int32),
        pltpu.VMEM((EB,), jnp.int32),
        pltpu.VMEM((EB,), jnp.int32),
        pltpu.VMEM((EB,), jnp.int32),
        pltpu.VMEM((EB, 16), jnp.float32),
        pltpu.VMEM((EB, 16), jnp.float32),
        pltpu.VMEM((EB, DH), jnp.float32),
        pltpu.VMEM((EB, DH), jnp.float32),
        pltpu.SemaphoreType.DMA,
        pltpu.SemaphoreType.DMA,
    ],
)(_mp_body)


RB = 400


def _dinv(deg_blk):
    d = deg_blk + 1.0
    return jnp.where(d > 0, lax.rsqrt(d), 0.0)


def _mm1_body(x_ref, w_ref, dA_ref, dB_ref, hA_ref, hB_ref, dinv_ref):
    dinv = _dinv(dA_ref[...] + dB_ref[...])
    h = jnp.dot(x_ref[...], w_ref[...],
                preferred_element_type=jnp.float32) * dinv
    hA_ref[...] = h[:, :DH]
    hB_ref[...] = h[:, DH:]
    dinv_ref[...] = dinv


def _mm2_body(sA_ref, sB_ref, hA_ref, hB_ref, dinv_ref, b1_ref, w2_ref,
              h2A_ref, h2B_ref):
    dinv = dinv_ref[...]
    o = jnp.concatenate(
        [sA_ref[...] + hA_ref[...], sB_ref[...] + hB_ref[...]], axis=1)
    o = jnp.maximum(o * dinv + b1_ref[...], 0.0)
    h2 = jnp.dot(o, w2_ref[...], preferred_element_type=jnp.float32) * dinv
    h2A_ref[...] = h2[:, :DH]
    h2B_ref[...] = h2[:, DH:]


def _mm3_body(sA_ref, sB_ref, hA_ref, hB_ref, dinv_ref, b2_ref, out_ref):
    o = jnp.concatenate(
        [sA_ref[...] + hA_ref[...], sB_ref[...] + hB_ref[...]], axis=1)
    out_ref[...] = o * dinv_ref[...] + b2_ref[...]


_half = pl.BlockSpec((RB, DH), lambda i: (i, 0))
_full = pl.BlockSpec((RB, D), lambda i: (i, 0))
_degs = pl.BlockSpec((RB, 1), lambda i: (i, 0))
_wspec = pl.BlockSpec((D, D), lambda i: (0, 0))
_bspec = pl.BlockSpec((1, D), lambda i: (0, 0))

_mm1 = pl.pallas_call(
    _mm1_body,
    grid=(N // RB,),
    in_specs=[_full, _wspec, _degs, _degs],
    out_specs=[_half, _half, _degs],
    out_shape=[jax.ShapeDtypeStruct((N, DH), jnp.float32),
               jax.ShapeDtypeStruct((N, DH), jnp.float32),
               jax.ShapeDtypeStruct((N, 1), jnp.float32)],
)

_mm2 = pl.pallas_call(
    _mm2_body,
    grid=(N // RB,),
    in_specs=[_half, _half, _half, _half, _degs, _bspec, _wspec],
    out_specs=[_half, _half],
    out_shape=[jax.ShapeDtypeStruct((N, DH), jnp.float32),
               jax.ShapeDtypeStruct((N, DH), jnp.float32)],
)

_mm3 = pl.pallas_call(
    _mm3_body,
    grid=(N // RB,),
    in_specs=[_half, _half, _half, _half, _degs, _bspec],
    out_specs=_full,
    out_shape=jax.ShapeDtypeStruct((N, D), jnp.float32),
)


@jax.jit
def kernel(x, edge_index, edge_weight, W1, b1, W2, b2):
    src = edge_index[0].astype(jnp.int32)
    dst = edge_index[1].astype(jnp.int32)
    pad = E_PAD - src.shape[0]
    src = jnp.concatenate([src, jnp.zeros((pad,), jnp.int32)])
    dst = jnp.concatenate([dst, jnp.zeros((pad,), jnp.int32)])
    ew = jnp.concatenate([edge_weight, jnp.zeros((pad,), jnp.float32)])

    zer1 = jnp.zeros((N,), jnp.float32)

    ew_rep = jnp.broadcast_to(ew[:, None], (E_PAD, 16))

    degA, degB = _deg_kernel(zer1, dst, ew)
    b1r = b1.reshape(1, D)
    b2r = b2.reshape(1, D)

    hA, hB, dinv = _mm1(x, W1, degA.reshape(N, 1), degB.reshape(N, 1))
    sA, sB = _mp_kernel(hA, hB, src, dst, ew_rep)
    h2A, h2B = _mm2(sA, sB, hA, hB, dinv, b1r, W2)
    s2A, s2B = _mp_kernel(h2A, h2B, src, dst, ew_rep)
    return _mm3(s2A, s2B, h2A, h2B, dinv, b2r)


# speedup vs baseline: 1.4050x; 1.0508x over previous
import functools

import jax
import jax.numpy as jnp
from jax import lax
from jax.experimental import pallas as pl
from jax.experimental.pallas import tpu as pltpu
from jax.experimental.pallas import tpu_sc as plsc

N = 10000
D = 256
DH = 128
NT = 16
EB = 64
E_PAD = 163840
EPT = E_PAD // NT
NB = EPT // EB
RPT = 632
RPT_LAST = N - 15 * RPT

_sc_mesh = plsc.VectorSubcoreMesh(core_axis_name="c", subcore_axis_name="s")


def _deg_body(zer_hbm, dst_hbm, ew_hbm, degA_hbm, degB_hbm, dacc, idxv, ewv):
    cid = lax.axis_index("c")
    sid = lax.axis_index("s")

    @pl.when(sid == 0)
    def _():
        pltpu.sync_copy(zer_hbm, dacc)

    plsc.subcore_barrier()

    def body(g, carry):
        off = cid * (E_PAD // 2) + sid * (EPT // 2) + g * EB
        pltpu.sync_copy(dst_hbm.at[pl.ds(off, EB)], idxv)
        pltpu.sync_copy(ew_hbm.at[pl.ds(off, EB)], ewv)
        pltpu.sync_copy(ewv, dacc.at[idxv], add=True)
        return carry

    lax.fori_loop(0, NB // 2, body, 0)
    plsc.subcore_barrier()

    @pl.when((cid == 0) & (sid == 0))
    def _():
        pltpu.sync_copy(dacc, degA_hbm)

    @pl.when((cid == 1) & (sid == 0))
    def _():
        pltpu.sync_copy(dacc, degB_hbm)


_deg_kernel = functools.partial(
    pl.kernel,
    out_type=[jax.ShapeDtypeStruct((N,), jnp.float32),
              jax.ShapeDtypeStruct((N,), jnp.float32)],
    mesh=_sc_mesh,
    scratch_types=[
        pltpu.VMEM_SHARED((N,), jnp.float32),
        pltpu.VMEM((EB,), jnp.int32),
        pltpu.VMEM((EB,), jnp.float32),
    ],
)(_deg_body)


def _mp_body(hA_hbm, hB_hbm, src_hbm, dst_hbm, ew_hbm, sA_hbm,
             sB_hbm, acc, idx0, idx1, dst0, dst1, ew0, ew1, rows0, rows1,
             sem0, sem1):
    cid = lax.axis_index("c")
    sid = lax.axis_index("s")
    bufs = ((idx0, dst0, ew0, rows0, sem0), (idx1, dst1, ew1, rows1, sem1))

    base = pl.multiple_of(sid * RPT, 8)

    def zrow(r, c2):
        for k in range(8):
            rows0[r, pl.ds(k * 16, 16)] = jnp.zeros((16,), jnp.float32)
        return c2

    lax.fori_loop(0, EB, zrow, 0)
    for j in range(8):
        pltpu.sync_copy(rows0, acc.at[pl.ds(base + j * EB, EB)])

    @pl.when(sid < 15)
    def _():
        pltpu.sync_copy(rows0, acc.at[pl.ds(base + 8 * EB, EB)])
        pltpu.sync_copy(rows0.at[pl.ds(0, RPT - 9 * EB)],
                        acc.at[pl.ds(base + 9 * EB, RPT - 9 * EB)])

    @pl.when(sid == 15)
    def _():
        pltpu.sync_copy(rows0.at[pl.ds(0, RPT_LAST - 8 * EB)],
                        acc.at[pl.ds(15 * RPT + 8 * EB, RPT_LAST - 8 * EB)])

    plsc.subcore_barrier()

    def stage_issue(g, b):
        idxb, dstb, ewb, rowsb, semb = bufs[b]
        off = sid * EPT + g * EB
        pltpu.sync_copy(src_hbm.at[pl.ds(off, EB)], idxb)
        pltpu.sync_copy(dst_hbm.at[pl.ds(off, EB)], dstb)
        pltpu.sync_copy(ew_hbm.at[pl.ds(off, EB)], ewb)

        @pl.when(cid == 0)
        def _():
            pltpu.async_copy(hA_hbm.at[idxb], rowsb, semb)

        @pl.when(cid == 1)
        def _():
            pltpu.async_copy(hB_hbm.at[idxb], rowsb, semb)

    def segment(g, b):
        idxb, dstb, ewb, rowsb, semb = bufs[b]

        @pl.when(g + 1 < NB)
        def _():
            stage_issue(g + 1, 1 - b)

        pltpu.make_async_copy(hA_hbm.at[pl.ds(0, EB)], rowsb, semb).wait()

        def scale(r, c2):
            ewr = ewb[pl.ds(r, 1)][0]
            for k in range(8):
                sl = pl.ds(k * 16, 16)
                rowsb[r, sl] = rowsb[r, sl] * ewr
            return c2

        lax.fori_loop(0, EB, scale, 0)
        pltpu.sync_copy(rowsb, acc.at[dstb], add=True)

    stage_issue(0, 0)

    def body(p, carry):
        segment(2 * p, 0)
        segment(2 * p + 1, 1)
        return carry

    lax.fori_loop(0, NB // 2, body, 0)
    plsc.subcore_barrier()

    @pl.when((cid == 0) & (sid < 15))
    def _():
        pltpu.sync_copy(acc.at[pl.ds(base, RPT)], sA_hbm.at[pl.ds(base, RPT)])

    @pl.when((cid == 0) & (sid == 15))
    def _():
        pltpu.sync_copy(acc.at[pl.ds(15 * RPT, RPT_LAST)],
                        sA_hbm.at[pl.ds(15 * RPT, RPT_LAST)])

    @pl.when((cid == 1) & (sid < 15))
    def _():
        pltpu.sync_copy(acc.at[pl.ds(base, RPT)], sB_hbm.at[pl.ds(base, RPT)])

    @pl.when((cid == 1) & (sid == 15))
    def _():
        pltpu.sync_copy(acc.at[pl.ds(15 * RPT, RPT_LAST)],
                        sB_hbm.at[pl.ds(15 * RPT, RPT_LAST)])


_mp_kernel = functools.partial(
    pl.kernel,
    out_type=[jax.ShapeDtypeStruct((N, DH), jnp.float32),
              jax.ShapeDtypeStruct((N, DH), jnp.float32)],
    mesh=_sc_mesh,
    scratch_types=[
        pltpu.VMEM_SHARED((N, DH), jnp.float32),
        pltpu.VMEM((EB,), jnp.int32),
        pltpu.VMEM((EB,), jnp.int32),
        pltpu.VMEM((EB,), jnp.int32),
        pltpu.VMEM((EB,), jnp.int32),
        pltpu.VMEM((EB,), jnp.float32),
        pltpu.VMEM((EB,), jnp.float32),
        pltpu.VMEM((EB, DH), jnp.float32),
        pltpu.VMEM((EB, DH), jnp.float32),
        pltpu.SemaphoreType.DMA,
        pltpu.SemaphoreType.DMA,
    ],
)(_mp_body)


RB = 400


def _dinv(deg_blk):
    d = deg_blk + 1.0
    return jnp.where(d > 0, lax.rsqrt(d), 0.0)


def _mm1_body(x_ref, w_ref, dA_ref, dB_ref, hA_ref, hB_ref, dinv_ref):
    dinv = _dinv(dA_ref[...] + dB_ref[...])
    h = jnp.dot(x_ref[...], w_ref[...],
                preferred_element_type=jnp.float32) * dinv
    hA_ref[...] = h[:, :DH]
    hB_ref[...] = h[:, DH:]
    dinv_ref[...] = dinv


def _mm2_body(sA_ref, sB_ref, hA_ref, hB_ref, dinv_ref, b1_ref, w2_ref,
              h2A_ref, h2B_ref):
    dinv = dinv_ref[...]
    o = jnp.concatenate(
        [sA_ref[...] + hA_ref[...], sB_ref[...] + hB_ref[...]], axis=1)
    o = jnp.maximum(o * dinv + b1_ref[...], 0.0)
    h2 = jnp.dot(o, w2_ref[...], preferred_element_type=jnp.float32) * dinv
    h2A_ref[...] = h2[:, :DH]
    h2B_ref[...] = h2[:, DH:]


def _mm3_body(sA_ref, sB_ref, hA_ref, hB_ref, dinv_ref, b2_ref, out_ref):
    o = jnp.concatenate(
        [sA_ref[...] + hA_ref[...], sB_ref[...] + hB_ref[...]], axis=1)
    out_ref[...] = o * dinv_ref[...] + b2_ref[...]


_half = pl.BlockSpec((RB, DH), lambda i: (i, 0))
_full = pl.BlockSpec((RB, D), lambda i: (i, 0))
_degs = pl.BlockSpec((RB, 1), lambda i: (i, 0))
_wspec = pl.BlockSpec((D, D), lambda i: (0, 0))
_bspec = pl.BlockSpec((1, D), lambda i: (0, 0))

_mm1 = pl.pallas_call(
    _mm1_body,
    grid=(N // RB,),
    in_specs=[_full, _wspec, _degs, _degs],
    out_specs=[_half, _half, _degs],
    out_shape=[jax.ShapeDtypeStruct((N, DH), jnp.float32),
               jax.ShapeDtypeStruct((N, DH), jnp.float32),
               jax.ShapeDtypeStruct((N, 1), jnp.float32)],
)

_mm2 = pl.pallas_call(
    _mm2_body,
    grid=(N // RB,),
    in_specs=[_half, _half, _half, _half, _degs, _bspec, _wspec],
    out_specs=[_half, _half],
    out_shape=[jax.ShapeDtypeStruct((N, DH), jnp.float32),
               jax.ShapeDtypeStruct((N, DH), jnp.float32)],
)

_mm3 = pl.pallas_call(
    _mm3_body,
    grid=(N // RB,),
    in_specs=[_half, _half, _half, _half, _degs, _bspec],
    out_specs=_full,
    out_shape=jax.ShapeDtypeStruct((N, D), jnp.float32),
)


@jax.jit
def kernel(x, edge_index, edge_weight, W1, b1, W2, b2):
    src = edge_index[0].astype(jnp.int32)
    dst = edge_index[1].astype(jnp.int32)
    pad = E_PAD - src.shape[0]
    src = jnp.concatenate([src, jnp.zeros((pad,), jnp.int32)])
    dst = jnp.concatenate([dst, jnp.zeros((pad,), jnp.int32)])
    ew = jnp.concatenate([edge_weight, jnp.zeros((pad,), jnp.float32)])

    zer1 = jnp.zeros((N,), jnp.float32)

    degA, degB = _deg_kernel(zer1, dst, ew)
    b1r = b1.reshape(1, D)
    b2r = b2.reshape(1, D)

    hA, hB, dinv = _mm1(x, W1, degA.reshape(N, 1), degB.reshape(N, 1))
    sA, sB = _mp_kernel(hA, hB, src, dst, ew)
    h2A, h2B = _mm2(sA, sB, hA, hB, dinv, b1r, W2)
    s2A, s2B = _mp_kernel(h2A, h2B, src, dst, ew)
    return _mm3(s2A, s2B, h2A, h2B, dinv, b2r)


# speedup vs baseline: 1.4842x; 1.0564x over previous
import functools

import jax
import jax.numpy as jnp
from jax import lax
from jax.experimental import pallas as pl
from jax.experimental.pallas import tpu as pltpu
from jax.experimental.pallas import tpu_sc as plsc

N = 10000
D = 256
DH = 128
NT = 16
EB = 80
E_PAD = 163840
EPT = E_PAD // NT
NB = EPT // EB
RPT = 632
RPT_LAST = N - 15 * RPT

_sc_mesh = plsc.VectorSubcoreMesh(core_axis_name="c", subcore_axis_name="s")


def _deg_body(zer_hbm, dst_hbm, ew_hbm, degA_hbm, degB_hbm, dacc, idxv, ewv):
    cid = lax.axis_index("c")
    sid = lax.axis_index("s")

    @pl.when(sid == 0)
    def _():
        pltpu.sync_copy(zer_hbm, dacc)

    plsc.subcore_barrier()

    def body(g, carry):
        off = cid * (E_PAD // 2) + sid * (EPT // 2) + g * EB
        pltpu.sync_copy(dst_hbm.at[pl.ds(off, EB)], idxv)
        pltpu.sync_copy(ew_hbm.at[pl.ds(off, EB)], ewv)
        pltpu.sync_copy(ewv, dacc.at[idxv], add=True)
        return carry

    lax.fori_loop(0, NB // 2, body, 0)
    plsc.subcore_barrier()

    @pl.when((cid == 0) & (sid == 0))
    def _():
        pltpu.sync_copy(dacc, degA_hbm)

    @pl.when((cid == 1) & (sid == 0))
    def _():
        pltpu.sync_copy(dacc, degB_hbm)


_deg_kernel = functools.partial(
    pl.kernel,
    out_type=[jax.ShapeDtypeStruct((N,), jnp.float32),
              jax.ShapeDtypeStruct((N,), jnp.float32)],
    mesh=_sc_mesh,
    scratch_types=[
        pltpu.VMEM_SHARED((N,), jnp.float32),
        pltpu.VMEM((EB,), jnp.int32),
        pltpu.VMEM((EB,), jnp.float32),
    ],
)(_deg_body)


def _mp_body(hA_hbm, hB_hbm, src_hbm, dst_hbm, ew_hbm, sA_hbm,
             sB_hbm, acc, idx0, idx1, dst0, dst1, ew0, ew1, rows0, rows1,
             sem0, sem1):
    cid = lax.axis_index("c")
    sid = lax.axis_index("s")
    bufs = ((idx0, dst0, ew0, rows0, sem0), (idx1, dst1, ew1, rows1, sem1))

    base = pl.multiple_of(sid * RPT, 8)

    def zrow(r, c2):
        for k in range(8):
            rows0[r, pl.ds(k * 16, 16)] = jnp.zeros((16,), jnp.float32)
        return c2

    lax.fori_loop(0, EB, zrow, 0)
    nf, rf = RPT // EB, RPT % EB
    nl, rl = RPT_LAST // EB, RPT_LAST % EB
    for j in range(min(nf, nl)):
        pltpu.sync_copy(rows0, acc.at[pl.ds(base + j * EB, EB)])

    @pl.when(sid < 15)
    def _():
        for j in range(min(nf, nl), nf):
            pltpu.sync_copy(rows0, acc.at[pl.ds(base + j * EB, EB)])
        if rf:
            pltpu.sync_copy(rows0.at[pl.ds(0, rf)],
                            acc.at[pl.ds(base + nf * EB, rf)])

    @pl.when(sid == 15)
    def _():
        for j in range(min(nf, nl), nl):
            pltpu.sync_copy(rows0, acc.at[pl.ds(15 * RPT + j * EB, EB)])
        if rl:
            pltpu.sync_copy(rows0.at[pl.ds(0, rl)],
                            acc.at[pl.ds(15 * RPT + nl * EB, rl)])

    plsc.subcore_barrier()

    def stage_issue(g, b):
        idxb, dstb, ewb, rowsb, semb = bufs[b]
        off = sid * EPT + g * EB
        pltpu.sync_copy(src_hbm.at[pl.ds(off, EB)], idxb)
        pltpu.sync_copy(dst_hbm.at[pl.ds(off, EB)], dstb)
        pltpu.sync_copy(ew_hbm.at[pl.ds(off, EB)], ewb)

        @pl.when(cid == 0)
        def _():
            pltpu.async_copy(hA_hbm.at[idxb], rowsb, semb)

        @pl.when(cid == 1)
        def _():
            pltpu.async_copy(hB_hbm.at[idxb], rowsb, semb)

    def segment(g, b):
        idxb, dstb, ewb, rowsb, semb = bufs[b]

        @pl.when(g + 1 < NB)
        def _():
            stage_issue(g + 1, 1 - b)

        pltpu.make_async_copy(hA_hbm.at[pl.ds(0, EB)], rowsb, semb).wait()

        def scale(r, c2):
            ewr = ewb[pl.ds(r, 1)][0]
            for k in range(8):
                sl = pl.ds(k * 16, 16)
                rowsb[r, sl] = rowsb[r, sl] * ewr
            return c2

        lax.fori_loop(0, EB, scale, 0)
        pltpu.sync_copy(rowsb, acc.at[dstb], add=True)

    stage_issue(0, 0)

    def body(p, carry):
        segment(2 * p, 0)
        segment(2 * p + 1, 1)
        return carry

    lax.fori_loop(0, NB // 2, body, 0)
    plsc.subcore_barrier()

    @pl.when((cid == 0) & (sid < 15))
    def _():
        pltpu.sync_copy(acc.at[pl.ds(base, RPT)], sA_hbm.at[pl.ds(base, RPT)])

    @pl.when((cid == 0) & (sid == 15))
    def _():
        pltpu.sync_copy(acc.at[pl.ds(15 * RPT, RPT_LAST)],
                        sA_hbm.at[pl.ds(15 * RPT, RPT_LAST)])

    @pl.when((cid == 1) & (sid < 15))
    def _():
        pltpu.sync_copy(acc.at[pl.ds(base, RPT)], sB_hbm.at[pl.ds(base, RPT)])

    @pl.when((cid == 1) & (sid == 15))
    def _():
        pltpu.sync_copy(acc.at[pl.ds(15 * RPT, RPT_LAST)],
                        sB_hbm.at[pl.ds(15 * RPT, RPT_LAST)])


_mp_kernel = functools.partial(
    pl.kernel,
    out_type=[jax.ShapeDtypeStruct((N, DH), jnp.float32),
              jax.ShapeDtypeStruct((N, DH), jnp.float32)],
    mesh=_sc_mesh,
    scratch_types=[
        pltpu.VMEM_SHARED((N, DH), jnp.float32),
        pltpu.VMEM((EB,), jnp.int32),
        pltpu.VMEM((EB,), jnp.int32),
        pltpu.VMEM((EB,), jnp.int32),
        pltpu.VMEM((EB,), jnp.int32),
        pltpu.VMEM((EB,), jnp.float32),
        pltpu.VMEM((EB,), jnp.float32),
        pltpu.VMEM((EB, DH), jnp.float32),
        pltpu.VMEM((EB, DH), jnp.float32),
        pltpu.SemaphoreType.DMA,
        pltpu.SemaphoreType.DMA,
    ],
)(_mp_body)


RB = 400


def _dinv(deg_blk):
    d = deg_blk + 1.0
    return jnp.where(d > 0, lax.rsqrt(d), 0.0)


def _mm1_body(x_ref, w_ref, dA_ref, dB_ref, hA_ref, hB_ref, dinv_ref):
    dinv = _dinv(dA_ref[...] + dB_ref[...])
    h = jnp.dot(x_ref[...], w_ref[...],
                preferred_element_type=jnp.float32) * dinv
    hA_ref[...] = h[:, :DH]
    hB_ref[...] = h[:, DH:]
    dinv_ref[...] = dinv


def _mm2_body(sA_ref, sB_ref, hA_ref, hB_ref, dinv_ref, b1_ref, w2_ref,
              h2A_ref, h2B_ref):
    dinv = dinv_ref[...]
    o = jnp.concatenate(
        [sA_ref[...] + hA_ref[...], sB_ref[...] + hB_ref[...]], axis=1)
    o = jnp.maximum(o * dinv + b1_ref[...], 0.0)
    h2 = jnp.dot(o, w2_ref[...], preferred_element_type=jnp.float32) * dinv
    h2A_ref[...] = h2[:, :DH]
    h2B_ref[...] = h2[:, DH:]


def _mm3_body(sA_ref, sB_ref, hA_ref, hB_ref, dinv_ref, b2_ref, out_ref):
    o = jnp.concatenate(
        [sA_ref[...] + hA_ref[...], sB_ref[...] + hB_ref[...]], axis=1)
    out_ref[...] = o * dinv_ref[...] + b2_ref[...]


_half = pl.BlockSpec((RB, DH), lambda i: (i, 0))
_full = pl.BlockSpec((RB, D), lambda i: (i, 0))
_degs = pl.BlockSpec((RB, 1), lambda i: (i, 0))
_wspec = pl.BlockSpec((D, D), lambda i: (0, 0))
_bspec = pl.BlockSpec((1, D), lambda i: (0, 0))

_mm1 = pl.pallas_call(
    _mm1_body,
    grid=(N // RB,),
    in_specs=[_full, _wspec, _degs, _degs],
    out_specs=[_half, _half, _degs],
    out_shape=[jax.ShapeDtypeStruct((N, DH), jnp.float32),
               jax.ShapeDtypeStruct((N, DH), jnp.float32),
               jax.ShapeDtypeStruct((N, 1), jnp.float32)],
)

_mm2 = pl.pallas_call(
    _mm2_body,
    grid=(N // RB,),
    in_specs=[_half, _half, _half, _half, _degs, _bspec, _wspec],
    out_specs=[_half, _half],
    out_shape=[jax.ShapeDtypeStruct((N, DH), jnp.float32),
               jax.ShapeDtypeStruct((N, DH), jnp.float32)],
)

_mm3 = pl.pallas_call(
    _mm3_body,
    grid=(N // RB,),
    in_specs=[_half, _half, _half, _half, _degs, _bspec],
    out_specs=_full,
    out_shape=jax.ShapeDtypeStruct((N, D), jnp.float32),
)


@jax.jit
def kernel(x, edge_index, edge_weight, W1, b1, W2, b2):
    src = edge_index[0].astype(jnp.int32)
    dst = edge_index[1].astype(jnp.int32)
    pad = E_PAD - src.shape[0]
    src = jnp.concatenate([src, jnp.zeros((pad,), jnp.int32)])
    dst = jnp.concatenate([dst, jnp.zeros((pad,), jnp.int32)])
    ew = jnp.concatenate([edge_weight, jnp.zeros((pad,), jnp.float32)])

    zer1 = jnp.zeros((N,), jnp.float32)

    degA, degB = _deg_kernel(zer1, dst, ew)
    b1r = b1.reshape(1, D)
    b2r = b2.reshape(1, D)

    hA, hB, dinv = _mm1(x, W1, degA.reshape(N, 1), degB.reshape(N, 1))
    sA, sB = _mp_kernel(hA, hB, src, dst, ew)
    h2A, h2B = _mm2(sA, sB, hA, hB, dinv, b1r, W2)
    s2A, s2B = _mp_kernel(h2A, h2B, src, dst, ew)
    return _mm3(s2A, s2B, h2A, h2B, dinv, b2r)
